# Initial kernel scaffold; baseline (speedup 1.0000x reference)
import jax
import jax.numpy as jnp
from jax.experimental import pallas as pl


def kernel(xyz1, xyz2, points1, points2, W1, b1, g1, be1, W2, b2, g2, be2):
    raise NotImplementedError("write your pallas kernel here")


# speedup vs baseline: 27.3590x; 27.3590x over previous
import functools

import jax
import jax.numpy as jnp
from jax.experimental import pallas as pl
from jax.experimental.pallas import tpu as pltpu


def _stage1_body(x1_ref, x2t_ref, p1_ref, p2_ref, w1a_ref, w1b_ref, b1_ref,
                 h1_ref, stats_ref, *, blk, s):
    bi = pl.program_id(0)
    ni = pl.program_id(1)

    x1 = x1_ref[0]
    x2t = x2t_ref[0]
    cross = jnp.dot(x1, x2t, preferred_element_type=jnp.float32)
    n1 = jnp.sum(x1 * x1, axis=1, keepdims=True)
    n2 = jnp.sum(x2t * x2t, axis=0, keepdims=True)
    d = -2.0 * cross
    d = d + n1
    d = d + n2
    d = jnp.maximum(d, jnp.float32(0.0001))

    iota = jax.lax.broadcasted_iota(jnp.int32, (blk, s), 1)
    vals, idxs = [], []
    for k in range(3):
        m = jnp.min(d, axis=1, keepdims=True)
        cand = jnp.where(d == m, iota, jnp.int32(s))
        ik = jnp.min(cand, axis=1, keepdims=True)
        vals.append(m)
        idxs.append(ik)
        if k < 2:
            d = jnp.where(iota == ik, jnp.float32(jnp.inf), d)

    r = [1.0 / (v + jnp.float32(0.0001)) for v in vals]
    norm = r[0] + r[1] + r[2]
    w = [x / (norm + jnp.float32(0.0001)) for x in r]

    wmat = jnp.where(iota == idxs[0], w[0],
           jnp.where(iota == idxs[1], w[1],
           jnp.where(iota == idxs[2], w[2], jnp.float32(0.0))))

    interp = jnp.dot(wmat, p2_ref[0], preferred_element_type=jnp.float32)
    h1 = (jnp.dot(p1_ref[0], w1a_ref[...], preferred_element_type=jnp.float32)
          + jnp.dot(interp, w1b_ref[...], preferred_element_type=jnp.float32)
          + b1_ref[...])
    h1_ref[0] = h1

    @pl.when((bi == 0) & (ni == 0))
    def _init():
        stats_ref[...] = jnp.zeros_like(stats_ref)

    stats_ref[0:1, :] += jnp.sum(h1, axis=0, keepdims=True)
    stats_ref[1:2, :] += jnp.sum(h1 * h1, axis=0, keepdims=True)


def _stage2_body(h1_ref, sc1_ref, sh1_ref, w2_ref, b2_ref,
                 h2_ref, stats_ref):
    i = pl.program_id(0)
    h = h1_ref[...]
    h = jnp.maximum(h * sc1_ref[...] + sh1_ref[...], jnp.float32(0.0))
    h2 = jnp.dot(h, w2_ref[...], preferred_element_type=jnp.float32) + b2_ref[...]
    h2_ref[...] = h2

    @pl.when(i == 0)
    def _init():
        stats_ref[...] = jnp.zeros_like(stats_ref)

    stats_ref[0:1, :] += jnp.sum(h2, axis=0, keepdims=True)
    stats_ref[1:2, :] += jnp.sum(h2 * h2, axis=0, keepdims=True)


def _stage3_body(h2_ref, sc2_ref, sh2_ref, out_ref):
    out_ref[...] = jnp.maximum(
        h2_ref[...] * sc2_ref[...] + sh2_ref[...], jnp.float32(0.0))


def kernel(xyz1, xyz2, points1, points2, W1, b1, g1, be1, W2, b2, g2, be2):
    B, N, _ = xyz1.shape
    S = xyz2.shape[1]
    C1 = points1.shape[2]
    C2 = points2.shape[2]
    C = W1.shape[1]

    blk1 = 256 if N % 256 == 0 else N
    blk2 = 512 if (B * N) % 512 == 0 else B * N

    xyz1p = jnp.pad(xyz1, ((0, 0), (0, 0), (0, 5)))
    xyz2t = jnp.transpose(jnp.pad(xyz2, ((0, 0), (0, 0), (0, 5))),
                          (0, 2, 1))
    W1a = W1[:C1]
    W1b = W1[C1:]
    b1r = b1.reshape(1, C)
    b2r = b2.reshape(1, C)

    nblk = N // blk1
    h1, stats1 = pl.pallas_call(
        functools.partial(_stage1_body, blk=blk1, s=S),
        grid=(B, nblk),
        in_specs=[
            pl.BlockSpec((1, blk1, 8), lambda b, n: (b, n, 0)),
            pl.BlockSpec((1, 8, S), lambda b, n: (b, 0, 0)),
            pl.BlockSpec((1, blk1, C1), lambda b, n: (b, n, 0)),
            pl.BlockSpec((1, S, C2), lambda b, n: (b, 0, 0)),
            pl.BlockSpec((C1, C), lambda b, n: (0, 0)),
            pl.BlockSpec((C2, C), lambda b, n: (0, 0)),
            pl.BlockSpec((1, C), lambda b, n: (0, 0)),
        ],
        out_specs=[
            pl.BlockSpec((1, blk1, C), lambda b, n: (b, n, 0)),
            pl.BlockSpec((8, C), lambda b, n: (0, 0)),
        ],
        out_shape=[
            jax.ShapeDtypeStruct((B, N, C), jnp.float32),
            jax.ShapeDtypeStruct((8, C), jnp.float32),
        ],
        compiler_params=pltpu.CompilerParams(
            dimension_semantics=("arbitrary", "arbitrary")),
    )(xyz1p, xyz2t, points1, points2, W1a, W1b, b1r)

    cnt = jnp.float32(B * N)
    mean1 = stats1[0:1] / cnt
    var1 = stats1[1:2] / cnt - mean1 * mean1
    sc1 = g1.reshape(1, C) / jnp.sqrt(var1 + 1e-5)
    sh1 = be1.reshape(1, C) - mean1 * sc1

    h1f = h1.reshape(B * N, C)
    nblk2 = (B * N) // blk2
    h2, stats2 = pl.pallas_call(
        _stage2_body,
        grid=(nblk2,),
        in_specs=[
            pl.BlockSpec((blk2, C), lambda i: (i, 0)),
            pl.BlockSpec((1, C), lambda i: (0, 0)),
            pl.BlockSpec((1, C), lambda i: (0, 0)),
            pl.BlockSpec((C, C), lambda i: (0, 0)),
            pl.BlockSpec((1, C), lambda i: (0, 0)),
        ],
        out_specs=[
            pl.BlockSpec((blk2, C), lambda i: (i, 0)),
            pl.BlockSpec((8, C), lambda i: (0, 0)),
        ],
        out_shape=[
            jax.ShapeDtypeStruct((B * N, C), jnp.float32),
            jax.ShapeDtypeStruct((8, C), jnp.float32),
        ],
        compiler_params=pltpu.CompilerParams(
            dimension_semantics=("arbitrary",)),
    )(h1f, sc1, sh1, W2, b2r)

    mean2 = stats2[0:1] / cnt
    var2 = stats2[1:2] / cnt - mean2 * mean2
    sc2 = g2.reshape(1, C) / jnp.sqrt(var2 + 1e-5)
    sh2 = be2.reshape(1, C) - mean2 * sc2

    out = pl.pallas_call(
        _stage3_body,
        grid=(nblk2,),
        in_specs=[
            pl.BlockSpec((blk2, C), lambda i: (i, 0)),
            pl.BlockSpec((1, C), lambda i: (0, 0)),
            pl.BlockSpec((1, C), lambda i: (0, 0)),
        ],
        out_specs=pl.BlockSpec((blk2, C), lambda i: (i, 0)),
        out_shape=jax.ShapeDtypeStruct((B * N, C), jnp.float32),
    )(h2, sc2, sh2)

    return out.reshape(B, N, C)


# speedup vs baseline: 31.7601x; 1.1609x over previous
import functools

import jax
import jax.numpy as jnp
from jax.experimental import pallas as pl
from jax.experimental.pallas import tpu as pltpu


def _stage1_body(x1_ref, x2t_ref, p1_ref, p2_ref, w1a_ref, w1b_ref, b1_ref,
                 h1_ref, stats_ref, *, blk, s):
    bi = pl.program_id(0)
    ni = pl.program_id(1)

    x1 = x1_ref[0]
    x2t = x2t_ref[0]
    cross = jnp.dot(x1, x2t, preferred_element_type=jnp.float32)
    n1 = jnp.sum(x1 * x1, axis=1, keepdims=True)
    n2 = jnp.sum(x2t * x2t, axis=0, keepdims=True)
    d = -2.0 * cross
    d = d + n1
    d = d + n2
    d = jnp.maximum(d, jnp.float32(0.0001))

    ch = 128
    nch = s // ch
    m1 = d[:, 0:ch]
    inf = jnp.full((blk, ch), jnp.inf, jnp.float32)
    m2 = inf
    m3 = inf
    for c in range(1, nch):
        cv = d[:, c * ch:(c + 1) * ch]
        nm1 = jnp.minimum(m1, cv)
        pu = jnp.maximum(m1, cv)
        nm2 = jnp.minimum(m2, pu)
        pu2 = jnp.maximum(m2, pu)
        m3 = jnp.minimum(m3, pu2)
        m1, m2 = nm1, nm2
    off = 1
    while off < ch:
        r1 = pltpu.roll(m1, ch - off, 1)
        r2 = pltpu.roll(m2, ch - off, 1)
        r3 = pltpu.roll(m3, ch - off, 1)
        p = jnp.maximum(m1, r1)
        q = jnp.minimum(m2, r2)
        c1 = jnp.minimum(m1, r1)
        c2 = jnp.minimum(p, q)
        c3 = jnp.minimum(jnp.maximum(p, q), jnp.minimum(m3, r3))
        m1, m2, m3 = c1, c2, c3
        off *= 2
    v1 = m1[:, 0:1]
    v2 = m2[:, 0:1]
    v3 = m3[:, 0:1]

    r1w = 1.0 / (v1 + jnp.float32(0.0001))
    r2w = 1.0 / (v2 + jnp.float32(0.0001))
    r3w = 1.0 / (v3 + jnp.float32(0.0001))
    norm = r1w + r2w + r3w
    scale = 1.0 / (norm + jnp.float32(0.0001))
    w1 = r1w * scale
    w2 = r2w * scale
    w3 = r3w * scale

    wmat = jnp.where(d == v1, w1,
           jnp.where(d == v2, w2,
           jnp.where(d == v3, w3, jnp.float32(0.0)))).astype(jnp.bfloat16)

    interp = jnp.dot(wmat, p2_ref[0].astype(jnp.bfloat16),
                     preferred_element_type=jnp.float32)
    h1 = (jnp.dot(p1_ref[0], w1a_ref[...], preferred_element_type=jnp.float32)
          + jnp.dot(interp, w1b_ref[...], preferred_element_type=jnp.float32)
          + b1_ref[...])
    h1_ref[0] = h1

    @pl.when((bi == 0) & (ni == 0))
    def _init():
        stats_ref[...] = jnp.zeros_like(stats_ref)

    stats_ref[0:1, :] += jnp.sum(h1, axis=0, keepdims=True)
    stats_ref[1:2, :] += jnp.sum(h1 * h1, axis=0, keepdims=True)


def _stage2_body(h1_ref, sc1_ref, sh1_ref, w2_ref, b2_ref,
                 h2_ref, stats_ref):
    i = pl.program_id(0)
    h = h1_ref[...]
    h = jnp.maximum(h * sc1_ref[...] + sh1_ref[...], jnp.float32(0.0))
    h2 = jnp.dot(h, w2_ref[...], preferred_element_type=jnp.float32) + b2_ref[...]
    h2_ref[...] = h2

    @pl.when(i == 0)
    def _init():
        stats_ref[...] = jnp.zeros_like(stats_ref)

    stats_ref[0:1, :] += jnp.sum(h2, axis=0, keepdims=True)
    stats_ref[1:2, :] += jnp.sum(h2 * h2, axis=0, keepdims=True)


def _stage3_body(h2_ref, sc2_ref, sh2_ref, out_ref):
    out_ref[...] = jnp.maximum(
        h2_ref[...] * sc2_ref[...] + sh2_ref[...], jnp.float32(0.0))


def kernel(xyz1, xyz2, points1, points2, W1, b1, g1, be1, W2, b2, g2, be2):
    B, N, _ = xyz1.shape
    S = xyz2.shape[1]
    C1 = points1.shape[2]
    C2 = points2.shape[2]
    C = W1.shape[1]

    blk1 = 256 if N % 256 == 0 else N
    blk2 = 512 if (B * N) % 512 == 0 else B * N

    xyz1p = jnp.pad(xyz1, ((0, 0), (0, 0), (0, 5)))
    xyz2t = jnp.transpose(jnp.pad(xyz2, ((0, 0), (0, 0), (0, 5))),
                          (0, 2, 1))
    W1a = W1[:C1]
    W1b = W1[C1:]
    b1r = b1.reshape(1, C)
    b2r = b2.reshape(1, C)

    nblk = N // blk1
    h1, stats1 = pl.pallas_call(
        functools.partial(_stage1_body, blk=blk1, s=S),
        grid=(B, nblk),
        in_specs=[
            pl.BlockSpec((1, blk1, 8), lambda b, n: (b, n, 0)),
            pl.BlockSpec((1, 8, S), lambda b, n: (b, 0, 0)),
            pl.BlockSpec((1, blk1, C1), lambda b, n: (b, n, 0)),
            pl.BlockSpec((1, S, C2), lambda b, n: (b, 0, 0)),
            pl.BlockSpec((C1, C), lambda b, n: (0, 0)),
            pl.BlockSpec((C2, C), lambda b, n: (0, 0)),
            pl.BlockSpec((1, C), lambda b, n: (0, 0)),
        ],
        out_specs=[
            pl.BlockSpec((1, blk1, C), lambda b, n: (b, n, 0)),
            pl.BlockSpec((8, C), lambda b, n: (0, 0)),
        ],
        out_shape=[
            jax.ShapeDtypeStruct((B, N, C), jnp.float32),
            jax.ShapeDtypeStruct((8, C), jnp.float32),
        ],
        compiler_params=pltpu.CompilerParams(
            dimension_semantics=("arbitrary", "arbitrary")),
    )(xyz1p, xyz2t, points1, points2, W1a, W1b, b1r)

    cnt = jnp.float32(B * N)
    mean1 = stats1[0:1] / cnt
    var1 = stats1[1:2] / cnt - mean1 * mean1
    sc1 = g1.reshape(1, C) / jnp.sqrt(var1 + 1e-5)
    sh1 = be1.reshape(1, C) - mean1 * sc1

    h1f = h1.reshape(B * N, C)
    nblk2 = (B * N) // blk2
    h2, stats2 = pl.pallas_call(
        _stage2_body,
        grid=(nblk2,),
        in_specs=[
            pl.BlockSpec((blk2, C), lambda i: (i, 0)),
            pl.BlockSpec((1, C), lambda i: (0, 0)),
            pl.BlockSpec((1, C), lambda i: (0, 0)),
            pl.BlockSpec((C, C), lambda i: (0, 0)),
            pl.BlockSpec((1, C), lambda i: (0, 0)),
        ],
        out_specs=[
            pl.BlockSpec((blk2, C), lambda i: (i, 0)),
            pl.BlockSpec((8, C), lambda i: (0, 0)),
        ],
        out_shape=[
            jax.ShapeDtypeStruct((B * N, C), jnp.float32),
            jax.ShapeDtypeStruct((8, C), jnp.float32),
        ],
        compiler_params=pltpu.CompilerParams(
            dimension_semantics=("arbitrary",)),
    )(h1f, sc1, sh1, W2, b2r)

    mean2 = stats2[0:1] / cnt
    var2 = stats2[1:2] / cnt - mean2 * mean2
    sc2 = g2.reshape(1, C) / jnp.sqrt(var2 + 1e-5)
    sh2 = be2.reshape(1, C) - mean2 * sc2

    out = pl.pallas_call(
        _stage3_body,
        grid=(nblk2,),
        in_specs=[
            pl.BlockSpec((blk2, C), lambda i: (i, 0)),
            pl.BlockSpec((1, C), lambda i: (0, 0)),
            pl.BlockSpec((1, C), lambda i: (0, 0)),
        ],
        out_specs=pl.BlockSpec((blk2, C), lambda i: (i, 0)),
        out_shape=jax.ShapeDtypeStruct((B * N, C), jnp.float32),
    )(h2, sc2, sh2)

    return out.reshape(B, N, C)
